# baseline (speedup 1.0000x reference)
import functools

import jax
import jax.numpy as jnp
from jax import lax
from jax.experimental import pallas as pl
from jax.experimental.pallas import tpu as pltpu
from jax.experimental.pallas import tpu_sc as plsc

VOCAB = 2548
DIM = 1000
DIM_PAD = 1024
BATCH = 4096

CHUNK = 16
NBUF = 4


def _make_embedding_kernel():
    info = plsc.get_sparse_core_info()
    num_cores, num_subcores = info.num_cores, info.num_subcores
    num_workers = num_cores * num_subcores
    b_per_w = BATCH // num_workers
    nchunks = b_per_w // CHUNK

    mesh = plsc.VectorSubcoreMesh(core_axis_name="c", subcore_axis_name="s")

    @functools.partial(
        pl.kernel,
        mesh=mesh,
        out_type=jax.ShapeDtypeStruct((BATCH, DIM_PAD), jnp.float32),
        scratch_types=[
            pltpu.VMEM((b_per_w,), jnp.int32),
            [pltpu.VMEM((CHUNK, DIM_PAD), jnp.float32) for _ in range(NBUF)],
            [pltpu.SemaphoreType.DMA for _ in range(NBUF)],
            [pltpu.SemaphoreType.DMA for _ in range(NBUF)],
        ],
    )
    def emb(x_hbm, w_hbm, out_hbm, idx_v, bufs, gsems, wsems):
        wid = lax.axis_index("s") * num_cores + lax.axis_index("c")
        base = wid * b_per_w
        pltpu.sync_copy(x_hbm.at[pl.ds(base, b_per_w)], idx_v)

        def gather(c):
            return pltpu.async_copy(
                w_hbm.at[idx_v.at[pl.ds(c * CHUNK, CHUNK)]],
                bufs[c % NBUF],
                gsems[c % NBUF],
            )

        def write(c):
            return pltpu.async_copy(
                bufs[c % NBUF],
                out_hbm.at[pl.ds(base + c * CHUNK, CHUNK)],
                wsems[c % NBUF],
            )

        g, w = {}, {}
        lead = NBUF // 2
        for c in range(lead):
            g[c] = gather(c)
        for c in range(nchunks):
            if c >= lead:
                w[c - lead].wait()
            if c + lead < nchunks:
                g[c + lead] = gather(c + lead)
            g[c].wait()
            w[c] = write(c)
        for c in range(max(0, nchunks - lead), nchunks):
            w[c].wait()

    return emb


_emb = _make_embedding_kernel()


def _slice_body(inp_ref, out_ref):
    out_ref[...] = inp_ref[:, :DIM]


def _make_slicer():
    tm = 256
    return pl.pallas_call(
        _slice_body,
        grid=(BATCH // tm,),
        in_specs=[pl.BlockSpec((tm, DIM_PAD), lambda i: (i, 0))],
        out_specs=pl.BlockSpec((tm, DIM), lambda i: (i, 0)),
        out_shape=jax.ShapeDtypeStruct((BATCH, DIM), jnp.float32),
    )


_slice = _make_slicer()


def kernel(x, weight):
    w_pad = jnp.pad(weight, ((0, 0), (0, DIM_PAD - DIM)))
    return _slice(_emb(x.astype(jnp.int32), w_pad))


# speedup vs baseline: 1.5830x; 1.5830x over previous
import functools

import jax
import jax.numpy as jnp
from jax import lax
from jax.experimental import pallas as pl
from jax.experimental.pallas import tpu as pltpu
from jax.experimental.pallas import tpu_sc as plsc

VOCAB = 2548
VOCAB_PAD = 2552
DIM = 1000
BATCH = 4096

ROWS = 8
NCHUNKS = DIM // ROWS
LANES = 16


def _make_embedding_kernel():
    info = plsc.get_sparse_core_info()
    num_cores, num_subcores = info.num_cores, info.num_subcores
    num_workers = num_cores * num_subcores
    max_chunks = -(-NCHUNKS // num_workers)

    mesh = plsc.VectorSubcoreMesh(core_axis_name="c", subcore_axis_name="s")

    @functools.partial(
        pl.kernel,
        mesh=mesh,
        out_type=jax.ShapeDtypeStruct((DIM, BATCH), jnp.float32),
        scratch_types=[
            pltpu.VMEM((BATCH,), jnp.int32),
            pltpu.VMEM((ROWS, VOCAB), jnp.float32),
            pltpu.VMEM((ROWS * VOCAB_PAD,), jnp.float32),
            pltpu.VMEM((ROWS * BATCH,), jnp.float32),
            pltpu.SemaphoreType.DMA,
            pltpu.SemaphoreType.DMA,
            pltpu.SemaphoreType.DMA,
        ],
        compiler_params=pltpu.CompilerParams(needs_layout_passes=False),
    )
    def emb(x_hbm, wt_hbm, out_hbm, idx_v, in_t, in_f, out_f, gsem, dsem, wsem):
        wid = lax.axis_index("s") * num_cores + lax.axis_index("c")
        pltpu.sync_copy(x_hbm, idx_v)

        for k in range(max_chunks):
            c = wid + k * num_workers

            @pl.when(c < NCHUNKS)
            def _():
                base = c * ROWS
                pltpu.async_copy(
                    wt_hbm.at[pl.ds(base, ROWS)], in_t, gsem
                ).wait()

                def body(j, carry):
                    cols = idx_v[pl.ds(j * LANES, LANES)]
                    for r in range(ROWS):
                        v = plsc.load_gather(
                            in_t, [jnp.full((LANES,), r, jnp.int32), cols]
                        )
                        out_f[pl.ds(r * BATCH + j * LANES, LANES)] = v
                    return carry

                lax.fori_loop(0, BATCH // LANES, body, 0)
                for r in range(ROWS):
                    pltpu.async_copy(
                        out_f.at[pl.ds(r * BATCH, BATCH)],
                        out_hbm.at[base + r],
                        wsem,
                    ).wait()

    return emb


_emb = _make_embedding_kernel()


def kernel(x, weight):
    out_t = _emb(x.astype(jnp.int32), weight.T)
    return out_t.T


# speedup vs baseline: 3.6954x; 2.3345x over previous
import functools

import jax
import jax.numpy as jnp
from jax import lax
from jax.experimental import pallas as pl
from jax.experimental.pallas import tpu as pltpu
from jax.experimental.pallas import tpu_sc as plsc

VOCAB = 2548
DIM = 1000
BATCH = 4096

ROWS = 8
NCHUNKS = DIM // ROWS
LANES = 16


def _make_embedding_kernel():
    info = plsc.get_sparse_core_info()
    num_cores, num_subcores = info.num_cores, info.num_subcores
    num_workers = num_cores * num_subcores
    max_chunks = -(-NCHUNKS // num_workers)

    mesh = plsc.VectorSubcoreMesh(core_axis_name="c", subcore_axis_name="s")

    @functools.partial(
        pl.kernel,
        mesh=mesh,
        out_type=jax.ShapeDtypeStruct((DIM, BATCH), jnp.float32),
        scratch_types=[
            pltpu.VMEM((BATCH,), jnp.int32),
            [pltpu.VMEM((ROWS, VOCAB), jnp.float32) for _ in range(2)],
            [pltpu.VMEM((ROWS, BATCH), jnp.float32) for _ in range(2)],
            pltpu.SemaphoreType.DMA,
            [pltpu.SemaphoreType.DMA for _ in range(2)],
            [pltpu.SemaphoreType.DMA for _ in range(2)],
        ],
        compiler_params=pltpu.CompilerParams(needs_layout_passes=False),
    )
    def emb(x_hbm, wt_hbm, out_hbm, idx_v, in_ts, out_fs, xsem, gsems, wsems):
        wid = lax.axis_index("s") * num_cores + lax.axis_index("c")
        pltpu.async_copy(x_hbm, idx_v, xsem)

        rows_splat = [jnp.full((LANES,), r, jnp.int32) for r in range(ROWS)]

        def cval(k):
            return wid + k * num_workers

        def in_slab(k):
            return wt_hbm.at[pl.ds(cval(k) * ROWS, ROWS)]

        def out_slab(k):
            return out_hbm.at[pl.ds(cval(k) * ROWS, ROWS)]

        pltpu.async_copy(in_slab(0), in_ts[0], gsems[0])
        pltpu.make_async_copy(x_hbm, idx_v, xsem).wait()

        for k in range(max_chunks):

            @pl.when(cval(k) < NCHUNKS)
            def _():
                if k + 1 < max_chunks:

                    @pl.when(cval(k + 1) < NCHUNKS)
                    def __():
                        pltpu.async_copy(
                            in_slab(k + 1), in_ts[(k + 1) % 2], gsems[(k + 1) % 2]
                        )

                pltpu.make_async_copy(in_slab(k), in_ts[k % 2], gsems[k % 2]).wait()
                if k >= 2:
                    pltpu.make_async_copy(
                        out_fs[k % 2], out_slab(k - 2), wsems[k % 2]
                    ).wait()

                in_t = in_ts[k % 2]
                out_f = out_fs[k % 2]

                @plsc.parallel_loop(0, BATCH // LANES, unroll=4)
                def _gather(j):
                    cols = idx_v[pl.ds(j * LANES, LANES)]
                    for r in range(ROWS):
                        v = plsc.load_gather(in_t, [rows_splat[r], cols])
                        out_f[r, pl.ds(j * LANES, LANES)] = v

                pltpu.async_copy(out_f, out_slab(k), wsems[k % 2])

        for k in range(max(0, max_chunks - 2), max_chunks):

            @pl.when(cval(k) < NCHUNKS)
            def _():
                pltpu.make_async_copy(
                    out_fs[k % 2], out_slab(k), wsems[k % 2]
                ).wait()

    return emb


_emb = _make_embedding_kernel()


def kernel(x, weight):
    out_t = _emb(x.astype(jnp.int32), weight.T)
    return out_t.T
